# baseline (speedup 1.0000x reference)
import functools

import jax
import jax.numpy as jnp
from jax import lax
from jax.experimental import pallas as pl
from jax.experimental.pallas import tpu as pltpu
from jax.experimental.pallas import tpu_sc as plsc

N_NODES = 50000
DIM = 64
N_EDGES = 800000
HALF = 25000
NP = 50176
RB = 6272
CH = 128
GC = 3
NG = 134
NCH = NG * GC
EP = 16 * NCH * CH
GB = GC * CH
R_ACC = 25088
STRIPE = R_ACC // 16
TAIL = HALF - 15 * STRIPE
TRW = 88
DEG_T = 65536
DSTRIPE = DEG_T // 16

_mesh = plsc.VectorSubcoreMesh(core_axis_name="c", subcore_axis_name="s")
_cp = pltpu.CompilerParams(use_tc_tiling_on_sc=False)


@functools.partial(
    pl.kernel,
    out_type=jax.ShapeDtypeStruct((2, DEG_T), jnp.float32),
    mesh=_mesh,
    compiler_params=_cp,
    scratch_types=[
        pltpu.VMEM((DSTRIPE,), jnp.float32),
        pltpu.VMEM((GB,), jnp.int32),
        pltpu.VMEM((GB,), jnp.int32),
        pltpu.VMEM((GC, CH), jnp.int32),
        pltpu.VMEM((CH,), jnp.float32),
        pltpu.VMEM_SHARED((DEG_T,), jnp.float32),
        pltpu.SemaphoreType.DMA((2,)),
        pltpu.SemaphoreType.DMA((GC,)),
    ],
)
def _degree_kernel(e_hbm, deg_hbm, zbuf, iba, ibb, xbuf, ones, acc,
                   isem, ssem):
    c = lax.axis_index("c")
    s = lax.axis_index("s")

    @pl.loop(0, DSTRIPE // 16)
    def _(i):
        zbuf[pl.ds(i * 16, 16)] = jnp.zeros((16,), jnp.float32)

    for j in range(CH // 16):
        ones[pl.ds(j * 16, 16)] = jnp.ones((16,), jnp.float32)

    pltpu.sync_copy(zbuf, acc.at[pl.ds(s * DSTRIPE, DSTRIPE)])
    plsc.subcore_barrier()

    lane = lax.iota(jnp.int32, 16)
    tbase = s * (NCH * CH)

    def idx_src(g):
        return e_hbm.at[c, pl.ds(tbase + g * GB, GB)]

    pltpu.async_copy(idx_src(0), iba, isem.at[0])
    pltpu.async_copy(idx_src(1), ibb, isem.at[1])

    def do_group(p, g, ib, bank, first):
        pltpu.make_async_copy(idx_src(g), ib, isem.at[bank]).wait()
        gi = (g % 120) * CH
        for k in range(GC):
            if first:
                @pl.when(p > 0)
                def _():
                    pltpu.make_async_copy(
                        ones, acc.at[xbuf.at[k]], ssem.at[k]).wait()
            else:
                pltpu.make_async_copy(
                    ones, acc.at[xbuf.at[k]], ssem.at[k]).wait()
            for j in range(CH // 16):
                v = ib[pl.ds(k * CH + j * 16, 16)]
                ok = (v >= 0) & (v < N_NODES)
                trash = N_NODES + gi + (lane + j * 16)
                xbuf[k, pl.ds(j * 16, 16)] = jnp.where(ok, v, trash)
            pltpu.async_copy(ones, acc.at[xbuf.at[k]], ssem.at[k], add=True)
        @pl.when(p < NG // 2 - 1)
        def _():
            pltpu.async_copy(idx_src(g + 2), ib, isem.at[bank])

    @pl.loop(0, NG // 2)
    def _(p):
        do_group(p, 2 * p, iba, 0, True)
        do_group(p, 2 * p + 1, ibb, 1, False)

    for k in range(GC):
        pltpu.make_async_copy(ones, acc.at[xbuf.at[k]], ssem.at[k]).wait()

    plsc.subcore_barrier()
    pltpu.sync_copy(acc.at[pl.ds(s * DSTRIPE, DSTRIPE)], zbuf)
    pltpu.sync_copy(zbuf, deg_hbm.at[c, pl.ds(s * DSTRIPE, DSTRIPE)])


@functools.partial(
    pl.kernel,
    out_type=jax.ShapeDtypeStruct((NP, DIM), jnp.float32),
    mesh=_mesh,
    compiler_params=_cp,
    scratch_types=[
        pltpu.VMEM((GB,), jnp.int32),
        pltpu.VMEM((GB,), jnp.int32),
        pltpu.VMEM((GB,), jnp.int32),
        pltpu.VMEM((GB,), jnp.int32),
        pltpu.VMEM((GC, CH), jnp.int32),
        pltpu.VMEM((GC, CH, DIM), jnp.float32),
        pltpu.VMEM_SHARED((R_ACC, DIM), jnp.float32),
        pltpu.SemaphoreType.DMA((2,)),
        pltpu.SemaphoreType.DMA((2,)),
        pltpu.SemaphoreType.DMA((GC,)),
        pltpu.SemaphoreType.DMA((GC,)),
    ],
)
def _prop_kernel(p_hbm, e_hbm, y_hbm, ha, ta, hb, tb, xbuf, rows, acc,
                 hsem, tsem, gsem, ssem):
    c = lax.axis_index("c")
    s = lax.axis_index("s")

    @pl.loop(0, CH)
    def _(r):
        for q in range(DIM // 16):
            rows[0, r, pl.ds(q * 16, 16)] = jnp.zeros((16,), jnp.float32)

    @pl.loop(0, STRIPE // CH)
    def _(k):
        pltpu.sync_copy(rows.at[0], acc.at[pl.ds(s * STRIPE + k * CH, CH)])

    pltpu.sync_copy(rows.at[0, pl.ds(0, STRIPE - (STRIPE // CH) * CH)],
                    acc.at[pl.ds(s * STRIPE + (STRIPE // CH) * CH,
                                 STRIPE - (STRIPE // CH) * CH)])

    plsc.subcore_barrier()

    lane = lax.iota(jnp.int32, 16)
    lo = c * HALF
    tbase = s * (NCH * CH)

    def h_src(g):
        return e_hbm.at[0, pl.ds(tbase + g * GB, GB)]

    def t_src(g):
        return e_hbm.at[1, pl.ds(tbase + g * GB, GB)]

    pltpu.async_copy(h_src(0), ha, hsem.at[0])
    pltpu.async_copy(t_src(0), ta, tsem.at[0])
    pltpu.async_copy(h_src(1), hb, hsem.at[1])
    pltpu.async_copy(t_src(1), tb, tsem.at[1])

    def do_group(p, g, hban, tban, bank, first):
        pltpu.make_async_copy(h_src(g), hban, hsem.at[bank]).wait()
        pltpu.make_async_copy(t_src(g), tban, tsem.at[bank]).wait()
        trash = [HALF + (lane + (j * 16) % TRW + (g * 16) % TRW) % TRW
                 for j in range(CH // 16)]
        for k in range(GC):
            if first:
                @pl.when(p > 0)
                def _():
                    pltpu.make_async_copy(
                        rows.at[k], acc.at[xbuf.at[k]], ssem.at[k]).wait()
            else:
                pltpu.make_async_copy(
                    rows.at[k], acc.at[xbuf.at[k]], ssem.at[k]).wait()
            pltpu.async_copy(
                p_hbm.at[tban.at[pl.ds(k * CH, CH)]], rows.at[k],
                gsem.at[k])
        for k in range(GC):
            pltpu.make_async_copy(
                p_hbm.at[tban.at[pl.ds(k * CH, CH)]], rows.at[k],
                gsem.at[k]).wait()
            for j in range(CH // 16):
                v = hban[pl.ds(k * CH + j * 16, 16)] - lo
                ok = (v >= 0) & (v < HALF)
                xbuf[k, pl.ds(j * 16, 16)] = jnp.where(ok, v, trash[j])
            pltpu.async_copy(
                rows.at[k], acc.at[xbuf.at[k]], ssem.at[k], add=True)
        @pl.when(p < NG // 2 - 1)
        def _():
            pltpu.async_copy(h_src(g + 2), hban, hsem.at[bank])
            pltpu.async_copy(t_src(g + 2), tban, tsem.at[bank])

    @pl.loop(0, NG // 2)
    def _(p):
        do_group(p, 2 * p, ha, ta, 0, True)
        do_group(p, 2 * p + 1, hb, tb, 1, False)

    for k in range(GC):
        pltpu.make_async_copy(rows.at[k], acc.at[xbuf.at[k]],
                              ssem.at[k]).wait()

    plsc.subcore_barrier()

    gbase = c * HALF + s * STRIPE

    @pl.when(s < 15)
    def _():
        pltpu.sync_copy(acc.at[pl.ds(s * STRIPE, STRIPE)],
                        y_hbm.at[pl.ds(gbase, STRIPE)])

    @pl.when(s == 15)
    def _():
        pltpu.sync_copy(acc.at[pl.ds(s * STRIPE, TAIL)],
                        y_hbm.at[pl.ds(gbase, TAIL)])


def _s0_body(x_ref, dh_ref, dt_ref, p_ref, dhi_ref, dti_ref):
    dhi = lax.rsqrt(jnp.maximum(dh_ref[...], 1.0))
    dti = lax.rsqrt(jnp.maximum(dt_ref[...], 1.0))
    p_ref[...] = x_ref[...] * dti
    dhi_ref[...] = dhi
    dti_ref[...] = dti


def _s1_body(y_ref, x_ref, dhi_ref, dti_ref, p_ref, a_ref):
    x1 = dhi_ref[...] * y_ref[...]
    p_ref[...] = dti_ref[...] * x1
    a_ref[...] = 2.0 * x_ref[...] + 2.0 * x1


def _s2_body(a_ref, y_ref, dhi_ref, o_ref):
    o_ref[...] = (a_ref[...] + dhi_ref[...] * y_ref[...]) * (1.0 / 3.0)


_mat = pl.BlockSpec((RB, DIM), lambda i: (i, 0))
_col = pl.BlockSpec((RB, 1), lambda i: (i, 0))
_fmat = jax.ShapeDtypeStruct((NP, DIM), jnp.float32)
_fcol = jax.ShapeDtypeStruct((NP, 1), jnp.float32)

_s0 = pl.pallas_call(
    _s0_body, grid=(NP // RB,),
    in_specs=[_mat, _col, _col],
    out_specs=[_mat, _col, _col],
    out_shape=[_fmat, _fcol, _fcol],
)

_s1 = pl.pallas_call(
    _s1_body, grid=(NP // RB,),
    in_specs=[_mat, _mat, _col, _col],
    out_specs=[_mat, _mat],
    out_shape=[_fmat, _fmat],
)

_s2 = pl.pallas_call(
    _s2_body, grid=(NP // RB,),
    in_specs=[_mat, _mat, _col],
    out_specs=_mat,
    out_shape=_fmat,
)


def kernel(u_emb, i_emb, edge_index):
    e = edge_index.astype(jnp.int32)
    e = jnp.concatenate(
        [e, jnp.full((2, EP - N_EDGES), N_NODES, jnp.int32)], axis=1)
    x = jnp.concatenate([u_emb, i_emb], axis=0)
    x = jnp.pad(x, ((0, NP - N_NODES), (0, 0)))

    deg = _degree_kernel(e)
    degh = jnp.pad(deg[0, :N_NODES], (0, NP - N_NODES)).reshape(NP, 1)
    degt = jnp.pad(deg[1, :N_NODES], (0, NP - N_NODES)).reshape(NP, 1)

    p0, dhi, dti = _s0(x, degh, degt)
    y1 = _prop_kernel(p0, e)
    p1, acc1 = _s1(y1, x, dhi, dti)
    y2 = _prop_kernel(p1, e)
    out = _s2(acc1, y2, dhi)
    return out[:N_NODES]


# speedup vs baseline: 1.3445x; 1.3445x over previous
import functools

import jax
import jax.numpy as jnp
from jax import lax
from jax.experimental import pallas as pl
from jax.experimental.pallas import tpu as pltpu
from jax.experimental.pallas import tpu_sc as plsc

N_NODES = 50000
DIM = 64
N_EDGES = 800000
HALF = 25000
NP = 50176
RB = 6272
CH = 128
EP = 819200
EPP = EP // 32
PCH = 2560
CAP = 26624
CAPG = CAP // CH
GPL = CAP // 1024
R_ACC = 25088
STRIPE = R_ACC // 16
TAIL = HALF - 15 * STRIPE
TRW = 88
ROWB = CH * DIM * 4
DEG_T = 65536
DSTRIPE = DEG_T // 16
HGC = 4
HGB = HGC * CH
HNG = EP // (16 * HGB)

_mesh = plsc.VectorSubcoreMesh(core_axis_name="c", subcore_axis_name="s")
_cp = pltpu.CompilerParams(use_tc_tiling_on_sc=False)
_cpl = pltpu.CompilerParams(use_tc_tiling_on_sc=False,
                            needs_layout_passes=False)


@functools.partial(
    pl.kernel,
    out_type=jax.ShapeDtypeStruct((2, DEG_T), jnp.float32),
    mesh=_mesh,
    compiler_params=_cp,
    scratch_types=[
        pltpu.VMEM((DSTRIPE,), jnp.float32),
        pltpu.VMEM((HGB,), jnp.int32),
        pltpu.VMEM((HGB,), jnp.int32),
        pltpu.VMEM((HGC, CH), jnp.int32),
        pltpu.VMEM((CH,), jnp.float32),
        pltpu.VMEM_SHARED((DEG_T,), jnp.float32),
        pltpu.SemaphoreType.DMA((2,)),
        pltpu.SemaphoreType.DMA((HGC,)),
    ],
)
def _degree_kernel(e_hbm, deg_hbm, zbuf, iba, ibb, xbuf, ones, acc,
                   isem, ssem):
    c = lax.axis_index("c")
    s = lax.axis_index("s")

    @pl.loop(0, DSTRIPE // 16)
    def _(i):
        zbuf[pl.ds(i * 16, 16)] = jnp.zeros((16,), jnp.float32)

    for j in range(CH // 16):
        ones[pl.ds(j * 16, 16)] = jnp.ones((16,), jnp.float32)

    pltpu.sync_copy(zbuf, acc.at[pl.ds(s * DSTRIPE, DSTRIPE)])
    plsc.subcore_barrier()

    lane = lax.iota(jnp.int32, 16)
    tbase = s * (EP // 16)

    def idx_src(g):
        return e_hbm.at[c, pl.ds(tbase + g * HGB, HGB)]

    pltpu.async_copy(idx_src(0), iba, isem.at[0])
    pltpu.async_copy(idx_src(1), ibb, isem.at[1])

    def do_group(p, g, ib, bank):
        pltpu.make_async_copy(idx_src(g), ib, isem.at[bank]).wait()
        gi = (g % 120) * CH
        for k in range(HGC):
            for j in range(CH // 16):
                v = ib[pl.ds(k * CH + j * 16, 16)]
                ok = (v >= 0) & (v < N_NODES)
                trash = N_NODES + gi + (lane + j * 16)
                xbuf[k, pl.ds(j * 16, 16)] = jnp.where(ok, v, trash)
            pltpu.async_copy(ones, acc.at[xbuf.at[k]], ssem.at[k], add=True)
        for k in range(HGC):
            pltpu.make_async_copy(
                ones, acc.at[xbuf.at[k]], ssem.at[k]).wait()
        @pl.when(p < HNG // 2 - 1)
        def _():
            pltpu.async_copy(idx_src(g + 2), ib, isem.at[bank])

    @pl.loop(0, HNG // 2)
    def _(p):
        do_group(p, 2 * p, iba, 0)
        do_group(p, 2 * p + 1, ibb, 1)

    plsc.subcore_barrier()
    pltpu.sync_copy(acc.at[pl.ds(s * DSTRIPE, DSTRIPE)], zbuf)
    pltpu.sync_copy(zbuf, deg_hbm.at[c, pl.ds(s * DSTRIPE, DSTRIPE)])


@functools.partial(
    pl.kernel,
    out_type=[
        jax.ShapeDtypeStruct((64, CAP), jnp.int32),
        jax.ShapeDtypeStruct((64, CAP), jnp.int32),
        jax.ShapeDtypeStruct((32, 16), jnp.int32),
    ],
    mesh=_mesh,
    compiler_params=_cpl,
    scratch_types=[
        pltpu.VMEM((2, PCH), jnp.int32),
        pltpu.VMEM((2, PCH), jnp.int32),
        pltpu.VMEM((CAP + 16,), jnp.int32),
        pltpu.VMEM((CAP + 16,), jnp.int32),
        pltpu.VMEM((CAP + 16,), jnp.int32),
        pltpu.VMEM((CAP + 16,), jnp.int32),
        pltpu.VMEM((16,), jnp.int32),
        pltpu.SemaphoreType.DMA((2,)),
        pltpu.SemaphoreType.DMA((2,)),
    ],
)
def _partition_kernel(e_hbm, ph_hbm, pt_hbm, cn_hbm,
                      hin, tin, h0, t0, h1, t1, cbuf, hsem, tsem):
    c = lax.axis_index("c")
    s = lax.axis_index("s")
    w = 2 * s + c
    lane = lax.iota(jnp.int32, 16)

    def h_src(ch):
        return e_hbm.at[0, pl.ds(w * EPP + ch * PCH, PCH)]

    def t_src(ch):
        return e_hbm.at[1, pl.ds(w * EPP + ch * PCH, PCH)]

    pltpu.async_copy(h_src(0), hin.at[0], hsem.at[0])
    pltpu.async_copy(t_src(0), tin.at[0], tsem.at[0])

    p0 = jnp.int32(0)
    p1 = jnp.int32(0)
    for ch in range(EPP // PCH):
        b = ch % 2
        if ch + 1 < EPP // PCH:
            nb = (ch + 1) % 2
            pltpu.async_copy(h_src(ch + 1), hin.at[nb], hsem.at[nb])
            pltpu.async_copy(t_src(ch + 1), tin.at[nb], tsem.at[nb])
        pltpu.make_async_copy(h_src(ch), hin.at[b], hsem.at[b]).wait()
        pltpu.make_async_copy(t_src(ch), tin.at[b], tsem.at[b]).wait()

        def body(i, carry):
            q0, q1 = carry
            hv = hin[b, pl.ds(i * 16, 16)]
            tv = tin[b, pl.ds(i * 16, 16)]
            m0 = hv < HALF
            m1 = (hv >= HALF) & (hv < N_NODES)
            i0 = m0.astype(jnp.int32)
            i1 = m1.astype(jnp.int32)
            pos0 = q0 + jnp.cumsum(i0) - 1
            pos1 = q1 + jnp.cumsum(i1) - 1
            idx0 = jnp.where(m0, pos0, CAP + lane)
            idx1 = jnp.where(m1, pos1, CAP + lane)
            plsc.store_scatter(h0, [idx0], hv)
            plsc.store_scatter(t0, [idx0], tv)
            plsc.store_scatter(h1, [idx1], hv - HALF)
            plsc.store_scatter(t1, [idx1], tv)
            return q0 + jnp.sum(i0), q1 + jnp.sum(i1)

        p0, p1 = lax.fori_loop(0, PCH // 16, body, (p0, p1))

    tpad = jnp.full((16,), N_NODES, jnp.int32)
    for i in range(64):
        tr = HALF + (lane + (i * 16) % TRW) % TRW
        h0[pl.ds(p0 + i * 16, 16)] = tr
        t0[pl.ds(p0 + i * 16, 16)] = tpad
        h1[pl.ds(p1 + i * 16, 16)] = tr
        t1[pl.ds(p1 + i * 16, 16)] = tpad

    c0r = ((p0 + 1023) // 1024) * 1024
    c1r = ((p1 + 1023) // 1024) * 1024
    cvec = jnp.where(lane == 0, c0r, jnp.where(lane == 1, c1r, 0))
    cbuf[pl.ds(0, 16)] = cvec

    pltpu.sync_copy(h0.at[pl.ds(0, CAP)], ph_hbm.at[w])
    pltpu.sync_copy(t0.at[pl.ds(0, CAP)], pt_hbm.at[w])
    pltpu.sync_copy(h1.at[pl.ds(0, CAP)], ph_hbm.at[32 + w])
    pltpu.sync_copy(t1.at[pl.ds(0, CAP)], pt_hbm.at[32 + w])
    pltpu.sync_copy(cbuf, cn_hbm.at[w])


@functools.partial(
    pl.kernel,
    out_type=jax.ShapeDtypeStruct((NP, DIM), jnp.float32),
    mesh=_mesh,
    compiler_params=_cpl,
    scratch_types=[
        pltpu.VMEM((8, CH), jnp.int32),
        pltpu.VMEM((8, CH), jnp.int32),
        pltpu.VMEM((8, CH), jnp.int32),
        pltpu.VMEM((8, CH), jnp.int32),
        pltpu.VMEM((2, CH, DIM), jnp.float32),
        pltpu.VMEM((16,), jnp.int32),
        pltpu.VMEM_SHARED((R_ACC, DIM), jnp.float32),
        pltpu.SemaphoreType.DMA((2,)),
        pltpu.SemaphoreType.DMA((2,)),
        pltpu.SemaphoreType.DMA((2,)),
        pltpu.SemaphoreType.DMA((2,)),
    ],
)
def _prop_kernel(p_hbm, ph_hbm, pt_hbm, cn_hbm, y_hbm,
                 xa, xb, ta, tb, rows, cbuf, acc, xsem, tsem, gsem, ssem):
    c = lax.axis_index("c")
    s = lax.axis_index("s")

    @pl.loop(0, CH)
    def _(r):
        for q in range(DIM // 16):
            rows[0, r, pl.ds(q * 16, 16)] = jnp.zeros((16,), jnp.float32)

    @pl.loop(0, STRIPE // CH)
    def _(k):
        pltpu.sync_copy(rows.at[0], acc.at[pl.ds(s * STRIPE + k * CH, CH)])

    pltpu.sync_copy(rows.at[0, pl.ds(0, STRIPE - (STRIPE // CH) * CH)],
                    acc.at[pl.ds(s * STRIPE + (STRIPE // CH) * CH,
                                 STRIPE - (STRIPE // CH) * CH)])

    plsc.subcore_barrier()

    for li in range(2):
        w = 2 * s + li
        L = c * 32 + w
        pltpu.sync_copy(cn_hbm.at[w], cbuf)
        cv = cbuf[pl.ds(0, 16)]
        lane = lax.iota(jnp.int32, 16)
        ng = jnp.sum(jnp.where(lane == c, cv, 0)) // 1024

        def x_src(g):
            return ph_hbm.at[L, pl.ds(g * 8, 8)]

        def t_src(g):
            return pt_hbm.at[L, pl.ds(g * 8, 8)]

        def scat_wait(r, xb_, k):
            pltpu.make_async_copy(
                rows.at[r], acc.at[xb_.at[k]], ssem.at[r]).wait()

        def do_group(g, xb_, tb_, bank):
            pltpu.make_async_copy(x_src(g), xb_, xsem.at[bank]).wait()
            pltpu.make_async_copy(t_src(g), tb_, tsem.at[bank]).wait()
            for k in range(8):
                r = k % 2
                if k >= 2:
                    scat_wait(r, xb_, k - 2)
                pltpu.async_copy(p_hbm.at[tb_.at[k]], rows.at[r],
                                 gsem.at[r])
                if k >= 1:
                    rr = (k - 1) % 2
                    pltpu.make_async_copy(
                        p_hbm.at[tb_.at[k - 1]], rows.at[rr],
                        gsem.at[rr]).wait()
                    pltpu.async_copy(rows.at[rr], acc.at[xb_.at[k - 1]],
                                     ssem.at[rr], add=True)
            pltpu.make_async_copy(
                p_hbm.at[tb_.at[7]], rows.at[1], gsem.at[1]).wait()
            pltpu.async_copy(rows.at[1], acc.at[xb_.at[7]],
                             ssem.at[1], add=True)
            scat_wait(0, xb_, 6)
            scat_wait(1, xb_, 7)
            @pl.when(g + 2 < ng)
            def _():
                pltpu.async_copy(x_src(g + 2), xb_, xsem.at[bank])
                pltpu.async_copy(t_src(g + 2), tb_, tsem.at[bank])

        @pl.when(0 < ng)
        def _():
            pltpu.async_copy(x_src(0), xa, xsem.at[0])
            pltpu.async_copy(t_src(0), ta, tsem.at[0])

        @pl.when(1 < ng)
        def _():
            pltpu.async_copy(x_src(1), xb, xsem.at[1])
            pltpu.async_copy(t_src(1), tb, tsem.at[1])

        @pl.loop(0, GPL // 2)
        def _(pp):
            @pl.when(2 * pp < ng)
            def _():
                do_group(2 * pp, xa, ta, 0)

            @pl.when(2 * pp + 1 < ng)
            def _():
                do_group(2 * pp + 1, xb, tb, 1)

    plsc.subcore_barrier()

    gbase = c * HALF + s * STRIPE

    @pl.when(s < 15)
    def _():
        pltpu.sync_copy(acc.at[pl.ds(s * STRIPE, STRIPE)],
                        y_hbm.at[pl.ds(gbase, STRIPE)])

    @pl.when(s == 15)
    def _():
        pltpu.sync_copy(acc.at[pl.ds(s * STRIPE, TAIL)],
                        y_hbm.at[pl.ds(gbase, TAIL)])


def _s0_body(x_ref, dh_ref, dt_ref, p_ref, dhi_ref, dti_ref):
    dhi = lax.rsqrt(jnp.maximum(dh_ref[...], 1.0))
    dti = lax.rsqrt(jnp.maximum(dt_ref[...], 1.0))
    p_ref[...] = x_ref[...] * dti
    dhi_ref[...] = dhi
    dti_ref[...] = dti


def _s1_body(y_ref, x_ref, dhi_ref, dti_ref, p_ref, a_ref):
    x1 = dhi_ref[...] * y_ref[...]
    p_ref[...] = dti_ref[...] * x1
    a_ref[...] = 2.0 * x_ref[...] + 2.0 * x1


def _s2_body(a_ref, y_ref, dhi_ref, o_ref):
    o_ref[...] = (a_ref[...] + dhi_ref[...] * y_ref[...]) * (1.0 / 3.0)


_mat = pl.BlockSpec((RB, DIM), lambda i: (i, 0))
_col = pl.BlockSpec((RB, 1), lambda i: (i, 0))
_fmat = jax.ShapeDtypeStruct((NP, DIM), jnp.float32)
_fcol = jax.ShapeDtypeStruct((NP, 1), jnp.float32)

_s0 = pl.pallas_call(
    _s0_body, grid=(NP // RB,),
    in_specs=[_mat, _col, _col],
    out_specs=[_mat, _col, _col],
    out_shape=[_fmat, _fcol, _fcol],
)

_s1 = pl.pallas_call(
    _s1_body, grid=(NP // RB,),
    in_specs=[_mat, _mat, _col, _col],
    out_specs=[_mat, _mat],
    out_shape=[_fmat, _fmat],
)

_s2 = pl.pallas_call(
    _s2_body, grid=(NP // RB,),
    in_specs=[_mat, _mat, _col],
    out_specs=_mat,
    out_shape=_fmat,
)


def kernel(u_emb, i_emb, edge_index):
    e = edge_index.astype(jnp.int32)
    e = jnp.concatenate(
        [e, jnp.full((2, EP - N_EDGES), N_NODES, jnp.int32)], axis=1)
    x = jnp.concatenate([u_emb, i_emb], axis=0)
    x = jnp.pad(x, ((0, NP - N_NODES), (0, 0)))

    deg = _degree_kernel(e)
    ph, pt, cn = _partition_kernel(e)
    ph = ph.reshape(64, CAPG, CH)
    pt = pt.reshape(64, CAPG, CH)
    degh = jnp.pad(deg[0, :N_NODES], (0, NP - N_NODES)).reshape(NP, 1)
    degt = jnp.pad(deg[1, :N_NODES], (0, NP - N_NODES)).reshape(NP, 1)

    p0, dhi, dti = _s0(x, degh, degt)
    y1 = _prop_kernel(p0, ph, pt, cn)
    p1, acc1 = _s1(y1, x, dhi, dti)
    y2 = _prop_kernel(p1, ph, pt, cn)
    out = _s2(acc1, y2, dhi)
    return out[:N_NODES]


# speedup vs baseline: 1.6188x; 1.2041x over previous
import functools

import jax
import jax.numpy as jnp
from jax import lax
from jax.experimental import pallas as pl
from jax.experimental.pallas import tpu as pltpu
from jax.experimental.pallas import tpu_sc as plsc

N_NODES = 50000
DIM = 64
N_EDGES = 800000
HALF = 25000
NP = 50176
RB = 6272
CH = 128
EP = 819200
EPP = EP // 32
PCH = 2560
CAP = 26624
CAPG = CAP // CH
GPL = CAP // 1024
R_ACC = 25088
STRIPE = R_ACC // 16
TAIL = HALF - 15 * STRIPE
TRW = 88
ROWB = CH * DIM * 4
DEG_T = 65536
DSTRIPE = DEG_T // 16
HGC = 4
HGB = HGC * CH
HNG = EP // (16 * HGB)

_mesh = plsc.VectorSubcoreMesh(core_axis_name="c", subcore_axis_name="s")
_cp = pltpu.CompilerParams(use_tc_tiling_on_sc=False)
_cpl = pltpu.CompilerParams(use_tc_tiling_on_sc=False,
                            needs_layout_passes=False)


@functools.partial(
    pl.kernel,
    out_type=jax.ShapeDtypeStruct((2, DEG_T), jnp.float32),
    mesh=_mesh,
    compiler_params=_cp,
    scratch_types=[
        pltpu.VMEM((DSTRIPE,), jnp.float32),
        pltpu.VMEM((HGB,), jnp.int32),
        pltpu.VMEM((HGB,), jnp.int32),
        pltpu.VMEM((HGC, CH), jnp.int32),
        pltpu.VMEM((CH,), jnp.float32),
        pltpu.VMEM_SHARED((DEG_T,), jnp.float32),
        pltpu.SemaphoreType.DMA((2,)),
        pltpu.SemaphoreType.DMA((HGC,)),
    ],
)
def _degree_kernel(e_hbm, deg_hbm, zbuf, iba, ibb, xbuf, ones, acc,
                   isem, ssem):
    c = lax.axis_index("c")
    s = lax.axis_index("s")

    @pl.loop(0, DSTRIPE // 16)
    def _(i):
        zbuf[pl.ds(i * 16, 16)] = jnp.zeros((16,), jnp.float32)

    for j in range(CH // 16):
        ones[pl.ds(j * 16, 16)] = jnp.ones((16,), jnp.float32)

    pltpu.sync_copy(zbuf, acc.at[pl.ds(s * DSTRIPE, DSTRIPE)])
    plsc.subcore_barrier()

    lane = lax.iota(jnp.int32, 16)
    tbase = s * (EP // 16)

    def idx_src(g):
        return e_hbm.at[c, pl.ds(tbase + g * HGB, HGB)]

    pltpu.async_copy(idx_src(0), iba, isem.at[0])
    pltpu.async_copy(idx_src(1), ibb, isem.at[1])

    def do_group(p, g, ib, bank):
        pltpu.make_async_copy(idx_src(g), ib, isem.at[bank]).wait()
        gi = (g % 120) * CH
        for k in range(HGC):
            for j in range(CH // 16):
                v = ib[pl.ds(k * CH + j * 16, 16)]
                ok = (v >= 0) & (v < N_NODES)
                trash = N_NODES + gi + (lane + j * 16)
                xbuf[k, pl.ds(j * 16, 16)] = jnp.where(ok, v, trash)
            pltpu.async_copy(ones, acc.at[xbuf.at[k]], ssem.at[k], add=True)
        for k in range(HGC):
            pltpu.make_async_copy(
                ones, acc.at[xbuf.at[k]], ssem.at[k]).wait()
        @pl.when(p < HNG // 2 - 1)
        def _():
            pltpu.async_copy(idx_src(g + 2), ib, isem.at[bank])

    @pl.loop(0, HNG // 2)
    def _(p):
        do_group(p, 2 * p, iba, 0)
        do_group(p, 2 * p + 1, ibb, 1)

    plsc.subcore_barrier()
    pltpu.sync_copy(acc.at[pl.ds(s * DSTRIPE, DSTRIPE)], zbuf)
    pltpu.sync_copy(zbuf, deg_hbm.at[c, pl.ds(s * DSTRIPE, DSTRIPE)])


@functools.partial(
    pl.kernel,
    out_type=[
        jax.ShapeDtypeStruct((64, CAP), jnp.int32),
        jax.ShapeDtypeStruct((64, CAP), jnp.int32),
        jax.ShapeDtypeStruct((32, 16), jnp.int32),
    ],
    mesh=_mesh,
    compiler_params=_cpl,
    scratch_types=[
        pltpu.VMEM((2, PCH), jnp.int32),
        pltpu.VMEM((2, PCH), jnp.int32),
        pltpu.VMEM((CAP + 16,), jnp.int32),
        pltpu.VMEM((CAP + 16,), jnp.int32),
        pltpu.VMEM((CAP + 16,), jnp.int32),
        pltpu.VMEM((CAP + 16,), jnp.int32),
        pltpu.VMEM((16,), jnp.int32),
        pltpu.SemaphoreType.DMA((2,)),
        pltpu.SemaphoreType.DMA((2,)),
    ],
)
def _partition_kernel(e_hbm, ph_hbm, pt_hbm, cn_hbm,
                      hin, tin, h0, t0, h1, t1, cbuf, hsem, tsem):
    c = lax.axis_index("c")
    s = lax.axis_index("s")
    w = 2 * s + c
    lane = lax.iota(jnp.int32, 16)

    def h_src(ch):
        return e_hbm.at[0, pl.ds(w * EPP + ch * PCH, PCH)]

    def t_src(ch):
        return e_hbm.at[1, pl.ds(w * EPP + ch * PCH, PCH)]

    pltpu.async_copy(h_src(0), hin.at[0], hsem.at[0])
    pltpu.async_copy(t_src(0), tin.at[0], tsem.at[0])

    p0 = jnp.int32(0)
    p1 = jnp.int32(0)
    for ch in range(EPP // PCH):
        b = ch % 2
        if ch + 1 < EPP // PCH:
            nb = (ch + 1) % 2
            pltpu.async_copy(h_src(ch + 1), hin.at[nb], hsem.at[nb])
            pltpu.async_copy(t_src(ch + 1), tin.at[nb], tsem.at[nb])
        pltpu.make_async_copy(h_src(ch), hin.at[b], hsem.at[b]).wait()
        pltpu.make_async_copy(t_src(ch), tin.at[b], tsem.at[b]).wait()

        def body(i, carry):
            q0, q1 = carry
            hv = hin[b, pl.ds(i * 16, 16)]
            tv = tin[b, pl.ds(i * 16, 16)]
            m0 = hv < HALF
            m1 = (hv >= HALF) & (hv < N_NODES)
            i0 = m0.astype(jnp.int32)
            i1 = m1.astype(jnp.int32)
            pos0 = q0 + jnp.cumsum(i0) - 1
            pos1 = q1 + jnp.cumsum(i1) - 1
            idx0 = jnp.where(m0, pos0, CAP + lane)
            idx1 = jnp.where(m1, pos1, CAP + lane)
            plsc.store_scatter(h0, [idx0], hv)
            plsc.store_scatter(t0, [idx0], tv)
            plsc.store_scatter(h1, [idx1], hv - HALF)
            plsc.store_scatter(t1, [idx1], tv)
            return q0 + jnp.sum(i0), q1 + jnp.sum(i1)

        p0, p1 = lax.fori_loop(0, PCH // 16, body, (p0, p1))

    tpad = jnp.full((16,), N_NODES, jnp.int32)
    for i in range(64):
        tr = HALF + (lane + (i * 16) % TRW) % TRW
        h0[pl.ds(p0 + i * 16, 16)] = tr
        t0[pl.ds(p0 + i * 16, 16)] = tpad
        h1[pl.ds(p1 + i * 16, 16)] = tr
        t1[pl.ds(p1 + i * 16, 16)] = tpad

    c0r = ((p0 + 1023) // 1024) * 1024
    c1r = ((p1 + 1023) // 1024) * 1024
    cvec = jnp.where(lane == 0, c0r, jnp.where(lane == 1, c1r, 0))
    cbuf[pl.ds(0, 16)] = cvec

    pltpu.sync_copy(h0.at[pl.ds(0, CAP)], ph_hbm.at[w])
    pltpu.sync_copy(t0.at[pl.ds(0, CAP)], pt_hbm.at[w])
    pltpu.sync_copy(h1.at[pl.ds(0, CAP)], ph_hbm.at[32 + w])
    pltpu.sync_copy(t1.at[pl.ds(0, CAP)], pt_hbm.at[32 + w])
    pltpu.sync_copy(cbuf, cn_hbm.at[w])


@functools.partial(
    pl.kernel,
    out_type=jax.ShapeDtypeStruct((NP, DIM), jnp.float32),
    mesh=_mesh,
    compiler_params=_cpl,
    scratch_types=[
        pltpu.VMEM((8, CH), jnp.int32),
        pltpu.VMEM((8, CH), jnp.int32),
        pltpu.VMEM((8, CH), jnp.int32),
        pltpu.VMEM((8, CH), jnp.int32),
        pltpu.VMEM((2, CH, DIM), jnp.bfloat16),
        pltpu.VMEM((2, CH, DIM), jnp.float32),
        pltpu.VMEM((16,), jnp.int32),
        pltpu.VMEM_SHARED((R_ACC, DIM), jnp.float32),
        pltpu.SemaphoreType.DMA((2,)),
        pltpu.SemaphoreType.DMA((2,)),
        pltpu.SemaphoreType.DMA((2,)),
        pltpu.SemaphoreType.DMA((2,)),
    ],
)
def _prop_kernel(p_hbm, ph_hbm, pt_hbm, cn_hbm, y_hbm,
                 xa, xb, ta, tb, rows, frows, cbuf, acc,
                 xsem, tsem, gsem, ssem):
    c = lax.axis_index("c")
    s = lax.axis_index("s")

    @pl.loop(0, CH)
    def _(r):
        for q in range(DIM // 16):
            frows[0, r, pl.ds(q * 16, 16)] = jnp.zeros((16,), jnp.float32)

    @pl.loop(0, STRIPE // CH)
    def _(k):
        pltpu.sync_copy(frows.at[0], acc.at[pl.ds(s * STRIPE + k * CH, CH)])

    pltpu.sync_copy(frows.at[0, pl.ds(0, STRIPE - (STRIPE // CH) * CH)],
                    acc.at[pl.ds(s * STRIPE + (STRIPE // CH) * CH,
                                 STRIPE - (STRIPE // CH) * CH)])

    plsc.subcore_barrier()

    for li in range(2):
        w = 2 * s + li
        L = c * 32 + w
        pltpu.sync_copy(cn_hbm.at[w], cbuf)
        cv = cbuf[pl.ds(0, 16)]
        lane = lax.iota(jnp.int32, 16)
        ng = jnp.sum(jnp.where(lane == c, cv, 0)) // 1024

        def x_src(g):
            return ph_hbm.at[L, pl.ds(g * 8, 8)]

        def t_src(g):
            return pt_hbm.at[L, pl.ds(g * 8, 8)]

        lane = lax.iota(jnp.int32, 16)

        def scat_wait(r, xb_, k):
            pltpu.make_async_copy(
                frows.at[r], acc.at[xb_.at[k]], ssem.at[r]).wait()

        def convert(r):
            @pl.loop(0, CH)
            def _(rr):
                rv = jnp.zeros((16,), jnp.int32) + rr
                for g in range(2):
                    v = rows[r, rr, pl.ds(32 * g, 32)]
                    av, bv = plsc.unpack(
                        v, format=plsc.PackFormat.INTERLEAVED)
                    ca = 32 * g + 2 * lane
                    plsc.store_scatter(frows.at[r], [rv, ca], av)
                    plsc.store_scatter(frows.at[r], [rv, ca + 1], bv)

        def do_group(g, xb_, tb_, bank):
            pltpu.make_async_copy(x_src(g), xb_, xsem.at[bank]).wait()
            pltpu.make_async_copy(t_src(g), tb_, tsem.at[bank]).wait()
            pltpu.async_copy(p_hbm.at[tb_.at[0]], rows.at[0], gsem.at[0])
            pltpu.async_copy(p_hbm.at[tb_.at[1]], rows.at[1], gsem.at[1])
            for k in range(8):
                r = k % 2
                pltpu.make_async_copy(
                    p_hbm.at[tb_.at[k]], rows.at[r], gsem.at[r]).wait()
                if k >= 2:
                    scat_wait(r, xb_, k - 2)
                convert(r)
                pltpu.async_copy(frows.at[r], acc.at[xb_.at[k]],
                                 ssem.at[r], add=True)
                if k + 2 < 8:
                    pltpu.async_copy(p_hbm.at[tb_.at[k + 2]], rows.at[r],
                                     gsem.at[r])
            scat_wait(0, xb_, 6)
            scat_wait(1, xb_, 7)
            @pl.when(g + 2 < ng)
            def _():
                pltpu.async_copy(x_src(g + 2), xb_, xsem.at[bank])
                pltpu.async_copy(t_src(g + 2), tb_, tsem.at[bank])

        @pl.when(0 < ng)
        def _():
            pltpu.async_copy(x_src(0), xa, xsem.at[0])
            pltpu.async_copy(t_src(0), ta, tsem.at[0])

        @pl.when(1 < ng)
        def _():
            pltpu.async_copy(x_src(1), xb, xsem.at[1])
            pltpu.async_copy(t_src(1), tb, tsem.at[1])

        @pl.loop(0, GPL // 2)
        def _(pp):
            @pl.when(2 * pp < ng)
            def _():
                do_group(2 * pp, xa, ta, 0)

            @pl.when(2 * pp + 1 < ng)
            def _():
                do_group(2 * pp + 1, xb, tb, 1)

    plsc.subcore_barrier()

    gbase = c * HALF + s * STRIPE

    @pl.when(s < 15)
    def _():
        pltpu.sync_copy(acc.at[pl.ds(s * STRIPE, STRIPE)],
                        y_hbm.at[pl.ds(gbase, STRIPE)])

    @pl.when(s == 15)
    def _():
        pltpu.sync_copy(acc.at[pl.ds(s * STRIPE, TAIL)],
                        y_hbm.at[pl.ds(gbase, TAIL)])


def _s0_body(x_ref, dh_ref, dt_ref, p_ref, dhi_ref, dti_ref):
    dhi = lax.rsqrt(jnp.maximum(dh_ref[...], 1.0))
    dti = lax.rsqrt(jnp.maximum(dt_ref[...], 1.0))
    p_ref[...] = (x_ref[...] * dti).astype(jnp.bfloat16)
    dhi_ref[...] = dhi
    dti_ref[...] = dti


def _s1_body(y_ref, x_ref, dhi_ref, dti_ref, p_ref, a_ref):
    x1 = dhi_ref[...] * y_ref[...]
    p_ref[...] = (dti_ref[...] * x1).astype(jnp.bfloat16)
    a_ref[...] = 2.0 * x_ref[...] + 2.0 * x1


def _s2_body(a_ref, y_ref, dhi_ref, o_ref):
    o_ref[...] = (a_ref[...] + dhi_ref[...] * y_ref[...]) * (1.0 / 3.0)


_mat = pl.BlockSpec((RB, DIM), lambda i: (i, 0))
_col = pl.BlockSpec((RB, 1), lambda i: (i, 0))
_fmat = jax.ShapeDtypeStruct((NP, DIM), jnp.float32)
_bmat = jax.ShapeDtypeStruct((NP, DIM), jnp.bfloat16)
_fcol = jax.ShapeDtypeStruct((NP, 1), jnp.float32)

_s0 = pl.pallas_call(
    _s0_body, grid=(NP // RB,),
    in_specs=[_mat, _col, _col],
    out_specs=[_mat, _col, _col],
    out_shape=[_bmat, _fcol, _fcol],
)

_s1 = pl.pallas_call(
    _s1_body, grid=(NP // RB,),
    in_specs=[_mat, _mat, _col, _col],
    out_specs=[_mat, _mat],
    out_shape=[_bmat, _fmat],
)

_s2 = pl.pallas_call(
    _s2_body, grid=(NP // RB,),
    in_specs=[_mat, _mat, _col],
    out_specs=_mat,
    out_shape=_fmat,
)


def kernel(u_emb, i_emb, edge_index):
    e = edge_index.astype(jnp.int32)
    e = jnp.concatenate(
        [e, jnp.full((2, EP - N_EDGES), N_NODES, jnp.int32)], axis=1)
    x = jnp.concatenate([u_emb, i_emb], axis=0)
    x = jnp.pad(x, ((0, NP - N_NODES), (0, 0)))

    deg = _degree_kernel(e)
    ph, pt, cn = _partition_kernel(e)
    ph = ph.reshape(64, CAPG, CH)
    pt = pt.reshape(64, CAPG, CH)
    degh = jnp.pad(deg[0, :N_NODES], (0, NP - N_NODES)).reshape(NP, 1)
    degt = jnp.pad(deg[1, :N_NODES], (0, NP - N_NODES)).reshape(NP, 1)

    p0, dhi, dti = _s0(x, degh, degt)
    y1 = _prop_kernel(p0, ph, pt, cn)
    p1, acc1 = _s1(y1, x, dhi, dti)
    y2 = _prop_kernel(p1, ph, pt, cn)
    out = _s2(acc1, y2, dhi)
    return out[:N_NODES]


# speedup vs baseline: 1.6422x; 1.0144x over previous
import functools

import jax
import jax.numpy as jnp
from jax import lax
from jax.experimental import pallas as pl
from jax.experimental.pallas import tpu as pltpu
from jax.experimental.pallas import tpu_sc as plsc

N_NODES = 50000
DIM = 64
N_EDGES = 800000
HALF = 25000
NP = 50176
RB = 6272
CH = 128
EP = 819200
EPP = EP // 32
PCH = 2560
CAP = 26624
CAPG = CAP // CH
GPL = CAP // 1024
R_ACC = 25088
STRIPE = R_ACC // 16
TAIL = HALF - 15 * STRIPE
TRW = 88
ROWB = CH * DIM * 4
DEG_T = 65536
DSTRIPE = DEG_T // 16
HGC = 4
HGB = HGC * CH
HNG = EP // (16 * HGB)

_mesh = plsc.VectorSubcoreMesh(core_axis_name="c", subcore_axis_name="s")
_cp = pltpu.CompilerParams(use_tc_tiling_on_sc=False)
_cpl = pltpu.CompilerParams(use_tc_tiling_on_sc=False,
                            needs_layout_passes=False)


@functools.partial(
    pl.kernel,
    out_type=jax.ShapeDtypeStruct((2, DEG_T), jnp.float32),
    mesh=_mesh,
    compiler_params=_cp,
    scratch_types=[
        pltpu.VMEM((DSTRIPE,), jnp.float32),
        pltpu.VMEM((HGB,), jnp.int32),
        pltpu.VMEM((HGB,), jnp.int32),
        pltpu.VMEM((HGC, CH), jnp.int32),
        pltpu.VMEM((CH,), jnp.float32),
        pltpu.VMEM_SHARED((DEG_T,), jnp.float32),
        pltpu.SemaphoreType.DMA((2,)),
        pltpu.SemaphoreType.DMA((HGC,)),
    ],
)
def _degree_kernel(e_hbm, deg_hbm, zbuf, iba, ibb, xbuf, ones, acc,
                   isem, ssem):
    c = lax.axis_index("c")
    s = lax.axis_index("s")

    @pl.loop(0, DSTRIPE // 16)
    def _(i):
        zbuf[pl.ds(i * 16, 16)] = jnp.zeros((16,), jnp.float32)

    for j in range(CH // 16):
        ones[pl.ds(j * 16, 16)] = jnp.ones((16,), jnp.float32)

    pltpu.sync_copy(zbuf, acc.at[pl.ds(s * DSTRIPE, DSTRIPE)])
    plsc.subcore_barrier()

    lane = lax.iota(jnp.int32, 16)
    tbase = s * (EP // 16)

    def idx_src(g):
        return e_hbm.at[c, pl.ds(tbase + g * HGB, HGB)]

    pltpu.async_copy(idx_src(0), iba, isem.at[0])
    pltpu.async_copy(idx_src(1), ibb, isem.at[1])

    def do_group(p, g, ib, bank):
        pltpu.make_async_copy(idx_src(g), ib, isem.at[bank]).wait()
        gi = (g % 120) * CH
        for k in range(HGC):
            for j in range(CH // 16):
                v = ib[pl.ds(k * CH + j * 16, 16)]
                ok = (v >= 0) & (v < N_NODES)
                trash = N_NODES + gi + (lane + j * 16)
                xbuf[k, pl.ds(j * 16, 16)] = jnp.where(ok, v, trash)
            pltpu.async_copy(ones, acc.at[xbuf.at[k]], ssem.at[k], add=True)
        for k in range(HGC):
            pltpu.make_async_copy(
                ones, acc.at[xbuf.at[k]], ssem.at[k]).wait()
        @pl.when(p < HNG // 2 - 1)
        def _():
            pltpu.async_copy(idx_src(g + 2), ib, isem.at[bank])

    @pl.loop(0, HNG // 2)
    def _(p):
        do_group(p, 2 * p, iba, 0)
        do_group(p, 2 * p + 1, ibb, 1)

    plsc.subcore_barrier()
    pltpu.sync_copy(acc.at[pl.ds(s * DSTRIPE, DSTRIPE)], zbuf)
    pltpu.sync_copy(zbuf, deg_hbm.at[c, pl.ds(s * DSTRIPE, DSTRIPE)])


@functools.partial(
    pl.kernel,
    out_type=[
        jax.ShapeDtypeStruct((64, CAP), jnp.int32),
        jax.ShapeDtypeStruct((64, CAP), jnp.int32),
        jax.ShapeDtypeStruct((32, 16), jnp.int32),
    ],
    mesh=_mesh,
    compiler_params=_cpl,
    scratch_types=[
        pltpu.VMEM((2, PCH), jnp.int32),
        pltpu.VMEM((2, PCH), jnp.int32),
        pltpu.VMEM((CAP + 16,), jnp.int32),
        pltpu.VMEM((CAP + 16,), jnp.int32),
        pltpu.VMEM((CAP + 16,), jnp.int32),
        pltpu.VMEM((CAP + 16,), jnp.int32),
        pltpu.VMEM((16,), jnp.int32),
        pltpu.SemaphoreType.DMA((2,)),
        pltpu.SemaphoreType.DMA((2,)),
    ],
)
def _partition_kernel(e_hbm, ph_hbm, pt_hbm, cn_hbm,
                      hin, tin, h0, t0, h1, t1, cbuf, hsem, tsem):
    c = lax.axis_index("c")
    s = lax.axis_index("s")
    w = 2 * s + c
    lane = lax.iota(jnp.int32, 16)

    def h_src(ch):
        return e_hbm.at[0, pl.ds(w * EPP + ch * PCH, PCH)]

    def t_src(ch):
        return e_hbm.at[1, pl.ds(w * EPP + ch * PCH, PCH)]

    pltpu.async_copy(h_src(0), hin.at[0], hsem.at[0])
    pltpu.async_copy(t_src(0), tin.at[0], tsem.at[0])

    p0 = jnp.int32(0)
    p1 = jnp.int32(0)
    for ch in range(EPP // PCH):
        b = ch % 2
        if ch + 1 < EPP // PCH:
            nb = (ch + 1) % 2
            pltpu.async_copy(h_src(ch + 1), hin.at[nb], hsem.at[nb])
            pltpu.async_copy(t_src(ch + 1), tin.at[nb], tsem.at[nb])
        pltpu.make_async_copy(h_src(ch), hin.at[b], hsem.at[b]).wait()
        pltpu.make_async_copy(t_src(ch), tin.at[b], tsem.at[b]).wait()

        def body(i, carry):
            q0, q1 = carry
            hv = hin[b, pl.ds(i * 16, 16)]
            tv = tin[b, pl.ds(i * 16, 16)]
            m0 = hv < HALF
            m1 = (hv >= HALF) & (hv < N_NODES)
            i0 = m0.astype(jnp.int32)
            i1 = m1.astype(jnp.int32)
            pos0 = q0 + jnp.cumsum(i0) - 1
            pos1 = q1 + jnp.cumsum(i1) - 1
            idx0 = jnp.where(m0, pos0, CAP + lane)
            idx1 = jnp.where(m1, pos1, CAP + lane)
            plsc.store_scatter(h0, [idx0], hv)
            plsc.store_scatter(t0, [idx0], tv)
            plsc.store_scatter(h1, [idx1], hv - HALF)
            plsc.store_scatter(t1, [idx1], tv)
            return q0 + jnp.sum(i0), q1 + jnp.sum(i1)

        p0, p1 = lax.fori_loop(0, PCH // 16, body, (p0, p1))

    tpad = jnp.full((16,), N_NODES, jnp.int32)
    for i in range(64):
        tr = HALF + (lane + (i * 16) % TRW) % TRW
        h0[pl.ds(p0 + i * 16, 16)] = tr
        t0[pl.ds(p0 + i * 16, 16)] = tpad
        h1[pl.ds(p1 + i * 16, 16)] = tr
        t1[pl.ds(p1 + i * 16, 16)] = tpad

    c0r = ((p0 + 1023) // 1024) * 1024
    c1r = ((p1 + 1023) // 1024) * 1024
    cvec = jnp.where(lane == 0, c0r, jnp.where(lane == 1, c1r, 0))
    cbuf[pl.ds(0, 16)] = cvec

    pltpu.sync_copy(h0.at[pl.ds(0, CAP)], ph_hbm.at[w])
    pltpu.sync_copy(t0.at[pl.ds(0, CAP)], pt_hbm.at[w])
    pltpu.sync_copy(h1.at[pl.ds(0, CAP)], ph_hbm.at[32 + w])
    pltpu.sync_copy(t1.at[pl.ds(0, CAP)], pt_hbm.at[32 + w])
    pltpu.sync_copy(cbuf, cn_hbm.at[w])


@functools.partial(
    pl.kernel,
    out_type=jax.ShapeDtypeStruct((NP, DIM), jnp.float32),
    mesh=_mesh,
    compiler_params=_cpl,
    scratch_types=[
        pltpu.VMEM((8, CH), jnp.int32),
        pltpu.VMEM((8, CH), jnp.int32),
        pltpu.VMEM((8, CH), jnp.int32),
        pltpu.VMEM((8, CH), jnp.int32),
        pltpu.VMEM((2, CH, DIM), jnp.bfloat16),
        pltpu.VMEM((2, CH, DIM), jnp.float32),
        pltpu.VMEM((16,), jnp.int32),
        pltpu.VMEM_SHARED((R_ACC, DIM), jnp.float32),
        pltpu.SemaphoreType.DMA((2,)),
        pltpu.SemaphoreType.DMA((2,)),
        pltpu.SemaphoreType.DMA((2,)),
        pltpu.SemaphoreType.DMA((2,)),
    ],
)
def _prop_kernel(p_hbm, ph_hbm, pt_hbm, cn_hbm, y_hbm,
                 xa, xb, ta, tb, rows, frows, cbuf, acc,
                 xsem, tsem, gsem, ssem):
    c = lax.axis_index("c")
    s = lax.axis_index("s")

    @pl.loop(0, CH)
    def _(r):
        for q in range(DIM // 16):
            frows[0, r, pl.ds(q * 16, 16)] = jnp.zeros((16,), jnp.float32)

    @pl.loop(0, STRIPE // CH)
    def _(k):
        pltpu.sync_copy(frows.at[0], acc.at[pl.ds(s * STRIPE + k * CH, CH)])

    pltpu.sync_copy(frows.at[0, pl.ds(0, STRIPE - (STRIPE // CH) * CH)],
                    acc.at[pl.ds(s * STRIPE + (STRIPE // CH) * CH,
                                 STRIPE - (STRIPE // CH) * CH)])

    plsc.subcore_barrier()

    for li in range(2):
        w = 2 * s + li
        L = c * 32 + w
        pltpu.sync_copy(cn_hbm.at[w], cbuf)
        cv = cbuf[pl.ds(0, 16)]
        lane = lax.iota(jnp.int32, 16)
        ng = jnp.sum(jnp.where(lane == c, cv, 0)) // 1024

        def x_src(g):
            return ph_hbm.at[L, pl.ds(g * 8, 8)]

        def t_src(g):
            return pt_hbm.at[L, pl.ds(g * 8, 8)]

        lane = lax.iota(jnp.int32, 16)
        ca = [g * 32 + 2 * lane + o for g in range(2) for o in range(2)]

        def scat_wait(r, xb_, k):
            pltpu.make_async_copy(
                frows.at[r], acc.at[xb_.at[k]], ssem.at[r]).wait()

        def convert(r):
            @pl.loop(0, CH // 4)
            def _(h4):
                for u in range(4):
                    rr = 4 * h4 + u
                    rv = jnp.zeros((16,), jnp.int32) + rr
                    for g in range(2):
                        v = rows[r, rr, pl.ds(32 * g, 32)]
                        av, bv = plsc.unpack(
                            v, format=plsc.PackFormat.INTERLEAVED)
                        plsc.store_scatter(frows.at[r], [rv, ca[2 * g]], av)
                        plsc.store_scatter(frows.at[r], [rv, ca[2 * g + 1]],
                                           bv)

        def do_group(g, xb_, tb_, bank):
            pltpu.make_async_copy(x_src(g), xb_, xsem.at[bank]).wait()
            pltpu.make_async_copy(t_src(g), tb_, tsem.at[bank]).wait()
            pltpu.async_copy(p_hbm.at[tb_.at[0]], rows.at[0], gsem.at[0])
            pltpu.async_copy(p_hbm.at[tb_.at[1]], rows.at[1], gsem.at[1])
            for k in range(8):
                r = k % 2
                pltpu.make_async_copy(
                    p_hbm.at[tb_.at[k]], rows.at[r], gsem.at[r]).wait()
                if k >= 2:
                    scat_wait(r, xb_, k - 2)
                convert(r)
                pltpu.async_copy(frows.at[r], acc.at[xb_.at[k]],
                                 ssem.at[r], add=True)
                if k + 2 < 8:
                    pltpu.async_copy(p_hbm.at[tb_.at[k + 2]], rows.at[r],
                                     gsem.at[r])
            scat_wait(0, xb_, 6)
            scat_wait(1, xb_, 7)
            @pl.when(g + 2 < ng)
            def _():
                pltpu.async_copy(x_src(g + 2), xb_, xsem.at[bank])
                pltpu.async_copy(t_src(g + 2), tb_, tsem.at[bank])

        @pl.when(0 < ng)
        def _():
            pltpu.async_copy(x_src(0), xa, xsem.at[0])
            pltpu.async_copy(t_src(0), ta, tsem.at[0])

        @pl.when(1 < ng)
        def _():
            pltpu.async_copy(x_src(1), xb, xsem.at[1])
            pltpu.async_copy(t_src(1), tb, tsem.at[1])

        @pl.loop(0, GPL // 2)
        def _(pp):
            @pl.when(2 * pp < ng)
            def _():
                do_group(2 * pp, xa, ta, 0)

            @pl.when(2 * pp + 1 < ng)
            def _():
                do_group(2 * pp + 1, xb, tb, 1)

    plsc.subcore_barrier()

    gbase = c * HALF + s * STRIPE

    @pl.when(s < 15)
    def _():
        pltpu.sync_copy(acc.at[pl.ds(s * STRIPE, STRIPE)],
                        y_hbm.at[pl.ds(gbase, STRIPE)])

    @pl.when(s == 15)
    def _():
        pltpu.sync_copy(acc.at[pl.ds(s * STRIPE, TAIL)],
                        y_hbm.at[pl.ds(gbase, TAIL)])


def _s0_body(x_ref, dh_ref, dt_ref, p_ref, dhi_ref, dti_ref):
    dhi = lax.rsqrt(jnp.maximum(dh_ref[...], 1.0))
    dti = lax.rsqrt(jnp.maximum(dt_ref[...], 1.0))
    p_ref[...] = (x_ref[...] * dti).astype(jnp.bfloat16)
    dhi_ref[...] = dhi
    dti_ref[...] = dti


def _s1_body(y_ref, x_ref, dhi_ref, dti_ref, p_ref, a_ref):
    x1 = dhi_ref[...] * y_ref[...]
    p_ref[...] = (dti_ref[...] * x1).astype(jnp.bfloat16)
    a_ref[...] = 2.0 * x_ref[...] + 2.0 * x1


def _s2_body(a_ref, y_ref, dhi_ref, o_ref):
    o_ref[...] = (a_ref[...] + dhi_ref[...] * y_ref[...]) * (1.0 / 3.0)


_mat = pl.BlockSpec((RB, DIM), lambda i: (i, 0))
_col = pl.BlockSpec((RB, 1), lambda i: (i, 0))
_fmat = jax.ShapeDtypeStruct((NP, DIM), jnp.float32)
_bmat = jax.ShapeDtypeStruct((NP, DIM), jnp.bfloat16)
_fcol = jax.ShapeDtypeStruct((NP, 1), jnp.float32)

_s0 = pl.pallas_call(
    _s0_body, grid=(NP // RB,),
    in_specs=[_mat, _col, _col],
    out_specs=[_mat, _col, _col],
    out_shape=[_bmat, _fcol, _fcol],
)

_s1 = pl.pallas_call(
    _s1_body, grid=(NP // RB,),
    in_specs=[_mat, _mat, _col, _col],
    out_specs=[_mat, _mat],
    out_shape=[_bmat, _fmat],
)

_s2 = pl.pallas_call(
    _s2_body, grid=(NP // RB,),
    in_specs=[_mat, _mat, _col],
    out_specs=_mat,
    out_shape=_fmat,
)


def kernel(u_emb, i_emb, edge_index):
    e = edge_index.astype(jnp.int32)
    e = jnp.concatenate(
        [e, jnp.full((2, EP - N_EDGES), N_NODES, jnp.int32)], axis=1)
    x = jnp.concatenate([u_emb, i_emb], axis=0)
    x = jnp.pad(x, ((0, NP - N_NODES), (0, 0)))

    deg = _degree_kernel(e)
    ph, pt, cn = _partition_kernel(e)
    ph = ph.reshape(64, CAPG, CH)
    pt = pt.reshape(64, CAPG, CH)
    degh = jnp.pad(deg[0, :N_NODES], (0, NP - N_NODES)).reshape(NP, 1)
    degt = jnp.pad(deg[1, :N_NODES], (0, NP - N_NODES)).reshape(NP, 1)

    p0, dhi, dti = _s0(x, degh, degt)
    y1 = _prop_kernel(p0, ph, pt, cn)
    p1, acc1 = _s1(y1, x, dhi, dti)
    y2 = _prop_kernel(p1, ph, pt, cn)
    out = _s2(acc1, y2, dhi)
    return out[:N_NODES]
